# baseline (speedup 1.0000x reference)
import functools

import jax
import jax.numpy as jnp
from jax import lax
from jax.experimental import pallas as pl
from jax.experimental.pallas import tpu as pltpu
from jax.experimental.pallas import tpu_sc as plsc

B = 16384
D = 32
DH = D // 2
NL = 100
NCAT = 1000
NC = 2
NS = 16
NW = NC * NS
BPW = B // NW
L = 16

PACK_TB = 16
PACK_BC = PACK_TB * 4 * 128


def _pack_body(xT_r, out_r):
    x = xT_r[...]
    for tt in range(PACK_TB):
        s = jnp.concatenate(
            [x[:, (tt * 4 + k) * 128:(tt * 4 + k + 1) * 128]
             for k in range(4)], axis=0)
        out_r[tt] = jnp.transpose(s)


def _pack(xT):
    n = xT.shape[1]
    nblk = pl.cdiv(n, PACK_BC)
    out = pl.pallas_call(
        _pack_body,
        grid=(nblk,),
        in_specs=[pl.BlockSpec((D, PACK_BC), lambda c: (0, c))],
        out_specs=pl.BlockSpec((PACK_TB, 128, 128), lambda c: (c, 0, 0)),
        out_shape=jax.ShapeDtypeStruct((nblk * PACK_TB, 128, 128),
                                       jnp.float32),
    )(xT)
    return jnp.reshape(out, (nblk * PACK_TB * 128, 128))


def _sc_gate(item, language, category, ltabT, ctabT,
             item_languages, item_categories, wl, bl, wc, bc):
    f32 = jnp.float32
    i32 = jnp.int32
    mesh = plsc.VectorSubcoreMesh(core_axis_name="c", subcore_axis_name="s")

    @functools.partial(
        pl.kernel,
        out_type=jax.ShapeDtypeStruct((B,), f32),
        mesh=mesh,
        compiler_params=pltpu.CompilerParams(use_tc_tiling_on_sc=True,
                                             needs_layout_passes=False),
        scratch_types=[
            pltpu.VMEM((BPW,), i32),
            pltpu.VMEM((BPW,), i32),
            pltpu.VMEM((BPW,), i32),
            pltpu.VMEM((BPW,), i32),
            pltpu.VMEM((BPW,), i32),
            pltpu.VMEM((DH, NL), f32),
            pltpu.VMEM((DH, NCAT), f32),
            pltpu.VMEM((DH,), f32),
            pltpu.VMEM((DH,), f32),
            pltpu.VMEM((L,), f32),
            pltpu.VMEM((L,), f32),
            pltpu.VMEM((BPW,), f32),
            pltpu.SemaphoreType.DMA,
        ],
    )
    def gate_kernel(item_h, lang_h, cat_h, ltabT_h, ctabT_h, ilang_h,
                    icat_h, wl_h, bl_h, wc_h, bc_h, gate_out,
                    iidx_v, lidx_v, cidx_v, ilidx_v, icidx_v,
                    ltab_v, ctab_v, wl_v, wc_v, bl_v, bc_v, gate_v, sem):
        wid = lax.axis_index("s") * NC + lax.axis_index("c")
        sl = pl.ds(wid * BPW, BPW)
        pltpu.sync_copy(item_h.at[sl], iidx_v)
        pltpu.sync_copy(lang_h.at[sl], lidx_v)
        pltpu.sync_copy(cat_h.at[sl], cidx_v)
        m1 = pltpu.async_copy(ilang_h.at[iidx_v], ilidx_v, sem)
        m2 = pltpu.async_copy(icat_h.at[iidx_v], icidx_v, sem)
        pltpu.sync_copy(ltabT_h, ltab_v)
        pltpu.sync_copy(ctabT_h, ctab_v)
        pltpu.sync_copy(wl_h, wl_v)
        pltpu.sync_copy(wc_h, wc_v)
        pltpu.sync_copy(bl_h, bl_v)
        pltpu.sync_copy(bc_h, bc_v)
        m1.wait()
        m2.wait()

        wlvec = wl_v[...]
        wcvec = wc_v[...]
        blvec = bl_v[...]
        bcvec = bc_v[...]

        def chunk_body(ci, _):
            r0 = ci * L
            lidx = lidx_v[pl.ds(r0, L)]
            ilidx = ilidx_v[pl.ds(r0, L)]
            cidx = cidx_v[pl.ds(r0, L)]
            icidx = icidx_v[pl.ds(r0, L)]
            acc_l = jnp.zeros((L,), f32)
            acc_c = jnp.zeros((L,), f32)
            for d in range(DH):
                drow = jnp.full((L,), d, i32)
                lv = plsc.load_gather(ltab_v, [drow, lidx])
                ilv = plsc.load_gather(ltab_v, [drow, ilidx])
                acc_l = acc_l + jnp.abs(lv - ilv) * wlvec[d]
                cv = plsc.load_gather(ctab_v, [drow, cidx])
                icv = plsc.load_gather(ctab_v, [drow, icidx])
                acc_c = acc_c + jnp.abs(cv - icv) * wcvec[d]
            sig_l = 1.0 / (1.0 + jnp.exp(-(acc_l + blvec)))
            sig_c = 1.0 / (1.0 + jnp.exp(-(acc_c + bcvec)))
            gate_v[pl.ds(r0, L)] = sig_l * sig_c
            return ()

        lax.fori_loop(0, BPW // L, chunk_body, (), unroll=1)
        pltpu.sync_copy(gate_v, gate_out.at[sl])

    return gate_kernel(item, language, category, ltabT, ctabT,
                       item_languages, item_categories, wl, bl, wc, bc)


def _sc_ui(idx, tab4):
    f32 = jnp.float32
    i32 = jnp.int32
    mesh = plsc.VectorSubcoreMesh(core_axis_name="c", subcore_axis_name="s")

    @functools.partial(
        pl.kernel,
        out_type=jax.ShapeDtypeStruct((D, B), f32),
        mesh=mesh,
        compiler_params=pltpu.CompilerParams(use_tc_tiling_on_sc=True,
                                             needs_layout_passes=False),
        scratch_types=[
            pltpu.VMEM((BPW,), i32),
            pltpu.VMEM((BPW,), i32),
            pltpu.VMEM((BPW, 128), f32),
            pltpu.VMEM((D, BPW), f32),
            pltpu.SemaphoreType.DMA,
        ],
    )
    def ui_kernel(idx_h, tab4_h, out_h, idx_v, q_v, x128_v, xT_v, sem):
        wid = lax.axis_index("s") * NC + lax.axis_index("c")
        sl = pl.ds(wid * BPW, BPW)
        pltpu.sync_copy(idx_h.at[sl], idx_v)

        def qbody(ci, _):
            s = pl.ds(ci * L, L)
            v = idx_v[s]
            q_v[s] = lax.shift_left(lax.shift_right_logical(v, 9), 7) \
                + (v & 127)
            return ()

        lax.fori_loop(0, BPW // L, qbody, (), unroll=4)

        pltpu.async_copy(tab4_h.at[q_v], x128_v, sem).wait()

        lane = lax.iota(i32, L)

        def ebody(ci, _):
            r0 = ci * L
            rows = r0 + lane
            basecol = (lax.shift_right_logical(idx_v[pl.ds(r0, L)], 7)
                       & 3) * D
            for d in range(D):
                xT_v[d, pl.ds(r0, L)] = \
                    plsc.load_gather(x128_v, [rows, basecol + d])
            return ()

        lax.fori_loop(0, BPW // L, ebody, (), unroll=1)
        pltpu.sync_copy(xT_v, out_h.at[:, sl])

    return ui_kernel(idx, tab4)


def _tc_dense(uT, iT, gate2d, W1uT, W1iT, b1c, W2T, b2c, w3c, b3):
    NB = 4096
    grid = (B // NB,)
    f32 = jnp.float32

    def body(uT_r, iT_r, gate_r, W1uT_r, W1iT_r, b1c_r, W2T_r, b2c_r,
             w3c_r, b3_r, out_r):
        h = jnp.dot(W1uT_r[...], uT_r[...], preferred_element_type=f32)
        h = h + jnp.dot(W1iT_r[...], iT_r[...], preferred_element_type=f32)
        h = jax.nn.relu(h + b1c_r[...])
        h = jax.nn.relu(jnp.dot(W2T_r[...], h, preferred_element_type=f32)
                        + b2c_r[...])
        base = jnp.sum(h * w3c_r[...], axis=0, keepdims=True) + b3_r[0, 0]
        out_r[...] = base * gate_r[...]

    colspec = lambda h: pl.BlockSpec((h, NB), lambda b: (0, b))
    full = lambda s: pl.BlockSpec(s, lambda b: (0,) * len(s))
    out = pl.pallas_call(
        body,
        grid=grid,
        in_specs=[
            colspec(D), colspec(D), colspec(1),
            full((128, D)), full((128, D)), full((128, 1)),
            full((64, 128)), full((64, 1)), full((64, 1)), full((1, 1)),
        ],
        out_specs=pl.BlockSpec((1, NB), lambda b: (0, b)),
        out_shape=jax.ShapeDtypeStruct((1, B), f32),
    )(uT, iT, gate2d, W1uT, W1iT, b1c, W2T, b2c, w3c, b3)
    return jnp.reshape(out, (B,))


def kernel(user, item, language, category, user_table, item_table,
           lang_table, cat_table, item_languages, item_categories,
           W_lang, b_lang, W_cat, b_cat, W1, b1, W2, b2, W3, b3):
    gate = _sc_gate(item, language, category, lang_table.T, cat_table.T,
                    item_languages, item_categories,
                    jnp.reshape(W_lang, (DH,)), jnp.broadcast_to(b_lang, (L,)),
                    jnp.reshape(W_cat, (DH,)), jnp.broadcast_to(b_cat, (L,)))
    utab4 = _pack(user_table.T)
    uT = _sc_ui(user, utab4)
    itab4 = _pack(item_table.T)
    iT = _sc_ui(item, itab4)
    gate2d = jnp.reshape(gate, (1, B))
    W1uT = jnp.transpose(W1[:D])
    W1iT = jnp.transpose(W1[D:])
    b1c = jnp.reshape(b1, (128, 1))
    W2T = jnp.transpose(W2)
    b2c = jnp.reshape(b2, (64, 1))
    w3c = jnp.reshape(W3, (64, 1))
    b3c = jnp.reshape(b3, (1, 1))
    return _tc_dense(uT, iT, gate2d, W1uT, W1iT, b1c, W2T, b2c, w3c, b3c)


# speedup vs baseline: 1.4004x; 1.4004x over previous
import functools

import jax
import jax.numpy as jnp
from jax import lax
from jax.experimental import pallas as pl
from jax.experimental.pallas import tpu as pltpu
from jax.experimental.pallas import tpu_sc as plsc

B = 16384
D = 32
DH = D // 2
NL = 100
NCAT = 1000
NTAB = 1000000
NC = 2
NS = 16
NW = NC * NS
BPW = B // NW
L = 16
PACK_TG = 8
PACK_BC = PACK_TG * 8 * 128


def _bfpack_piece(piece, even_sel, odd_sel):
    f32 = jnp.float32
    i32 = jnp.int32
    ev = lax.dot_general(even_sel, piece, (((1,), (0,)), ((), ())),
                         preferred_element_type=f32)
    od = lax.dot_general(odd_sel, piece, (((1,), (0,)), ((), ())),
                         preferred_element_type=f32)
    lo = lax.bitcast_convert_type(ev.astype(jnp.bfloat16), jnp.int16)
    hi = lax.bitcast_convert_type(od.astype(jnp.bfloat16), jnp.int16)
    lo32 = lo.astype(i32) & jnp.int32(0xFFFF)
    hi32 = lax.shift_left(hi.astype(i32), 16)
    return lo32 | hi32


def _pack_body(xT_r, yT_r, esel_r, osel_r, outx_r, outy_r):
    x = xT_r[...]
    y = yT_r[...]
    es = esel_r[...]
    os_ = osel_r[...]
    for g in range(PACK_TG):
        sx = jnp.concatenate(
            [_bfpack_piece(x[:, (g * 8 + k) * 128:(g * 8 + k + 1) * 128],
                           es, os_) for k in range(8)], axis=0)
        outx_r[g] = jnp.transpose(sx)
        sy = jnp.concatenate(
            [_bfpack_piece(y[:, (g * 8 + k) * 128:(g * 8 + k + 1) * 128],
                           es, os_) for k in range(8)], axis=0)
        outy_r[g] = jnp.transpose(sy)


def _pack2(xT, yT):
    n = xT.shape[1]
    nblk = pl.cdiv(n, PACK_BC)
    dd = jnp.arange(D, dtype=jnp.int32)
    pp = jnp.arange(DH, dtype=jnp.int32)
    even_sel = (dd[None, :] == 2 * pp[:, None]).astype(jnp.float32)
    odd_sel = (dd[None, :] == 2 * pp[:, None] + 1).astype(jnp.float32)
    out_t = jax.ShapeDtypeStruct((nblk * PACK_TG, 128, 128), jnp.int32)
    outx, outy = pl.pallas_call(
        _pack_body,
        grid=(nblk,),
        in_specs=[pl.BlockSpec((D, PACK_BC), lambda c: (0, c)),
                  pl.BlockSpec((D, PACK_BC), lambda c: (0, c)),
                  pl.BlockSpec((DH, D), lambda c: (0, 0)),
                  pl.BlockSpec((DH, D), lambda c: (0, 0))],
        out_specs=[pl.BlockSpec((PACK_TG, 128, 128), lambda c: (c, 0, 0)),
                   pl.BlockSpec((PACK_TG, 128, 128), lambda c: (c, 0, 0))],
        out_shape=[out_t, out_t],
    )(xT, yT, even_sel, odd_sel)
    m = nblk * PACK_TG * 128
    return jnp.reshape(outx, (m, 128)), jnp.reshape(outy, (m, 128))


def _sc_gather(user, item, language, category, utab4, itab4, ltabT, ctabT,
               item_languages, item_categories, wl, bl, wc, bc):
    f32 = jnp.float32
    i32 = jnp.int32
    mesh = plsc.VectorSubcoreMesh(core_axis_name="c", subcore_axis_name="s")

    @functools.partial(
        pl.kernel,
        out_type=[
            jax.ShapeDtypeStruct((D, B), f32),
            jax.ShapeDtypeStruct((D, B), f32),
            jax.ShapeDtypeStruct((B,), f32),
        ],
        mesh=mesh,
        compiler_params=pltpu.CompilerParams(use_tc_tiling_on_sc=True,
                                             needs_layout_passes=False),
        scratch_types=[
            pltpu.VMEM((BPW,), i32),
            pltpu.VMEM((BPW,), i32),
            pltpu.VMEM((BPW,), i32),
            pltpu.VMEM((BPW,), i32),
            pltpu.VMEM((BPW,), i32),
            pltpu.VMEM((BPW,), i32),
            pltpu.VMEM((BPW,), i32),
            pltpu.VMEM((BPW,), i32),
            pltpu.VMEM((BPW, 128), i32),
            pltpu.VMEM((D, BPW), f32),
            pltpu.VMEM((D, BPW), f32),
            pltpu.VMEM((DH, NL), f32),
            pltpu.VMEM((DH, NCAT), f32),
            pltpu.VMEM((DH,), f32),
            pltpu.VMEM((DH,), f32),
            pltpu.VMEM((L,), f32),
            pltpu.VMEM((L,), f32),
            pltpu.VMEM((BPW,), f32),
            pltpu.SemaphoreType.DMA,
            pltpu.SemaphoreType.DMA,
        ],
    )
    def sc_kernel(user_h, item_h, lang_h, cat_h, utab4_h, itab4_h, ltabT_h,
                  ctabT_h, ilang_h, icat_h, wl_h, bl_h, wc_h, bc_h,
                  uT_out, iT_out, gate_out,
                  uidx_v, iidx_v, lidx_v, cidx_v, ilidx_v, icidx_v,
                  uq_v, iq_v, x128_v, uT_v, iT_v, ltab_v, ctab_v,
                  wl_v, wc_v, bl_v, bc_v, gate_v, sem, sem2):
        wid = lax.axis_index("s") * NC + lax.axis_index("c")
        base = wid * BPW
        sl = pl.ds(base, BPW)
        pltpu.sync_copy(user_h.at[sl], uidx_v)
        pltpu.sync_copy(item_h.at[sl], iidx_v)
        pltpu.sync_copy(lang_h.at[sl], lidx_v)
        pltpu.sync_copy(cat_h.at[sl], cidx_v)
        m1 = pltpu.async_copy(ilang_h.at[iidx_v], ilidx_v, sem2)
        m2 = pltpu.async_copy(icat_h.at[iidx_v], icidx_v, sem2)
        pltpu.sync_copy(ltabT_h, ltab_v)
        pltpu.sync_copy(ctabT_h, ctab_v)
        pltpu.sync_copy(wl_h, wl_v)
        pltpu.sync_copy(wc_h, wc_v)
        pltpu.sync_copy(bl_h, bl_v)
        pltpu.sync_copy(bc_h, bc_v)

        def qbody(ci, _):
            s = pl.ds(ci * L, L)
            u = uidx_v[s]
            i = iidx_v[s]
            uq_v[s] = lax.shift_left(lax.shift_right_logical(u, 10), 7) \
                + (u & 127)
            iq_v[s] = lax.shift_left(lax.shift_right_logical(i, 10), 7) \
                + (i & 127)
            return ()

        lax.fori_loop(0, BPW // L, qbody, (), unroll=4)

        lane = lax.iota(i32, L)

        himask = jnp.int32(-65536)

        def extract(idx_ref, dst_ref):
            def ebody(ci, _):
                r0 = ci * L
                rows = r0 + lane
                basecol = (lax.shift_right_logical(idx_ref[pl.ds(r0, L)], 7)
                           & 7) * L
                for p in range(DH):
                    v = plsc.load_gather(x128_v, [rows, basecol + p])
                    dst_ref[2 * p, pl.ds(r0, L)] = \
                        plsc.bitcast(lax.shift_left(v, 16), f32)
                    dst_ref[2 * p + 1, pl.ds(r0, L)] = \
                        plsc.bitcast(v & himask, f32)
                return ()

            lax.fori_loop(0, BPW // L, ebody, (), unroll=1)

        pltpu.async_copy(utab4_h.at[uq_v], x128_v, sem).wait()
        extract(uidx_v, uT_v)
        pltpu.async_copy(itab4_h.at[iq_v], x128_v, sem).wait()
        extract(iidx_v, iT_v)

        m1.wait()
        m2.wait()

        wlvec = wl_v[...]
        wcvec = wc_v[...]
        blvec = bl_v[...]
        bcvec = bc_v[...]

        def chunk_body(ci, _):
            r0 = ci * L
            lidx = lidx_v[pl.ds(r0, L)]
            ilidx = ilidx_v[pl.ds(r0, L)]
            cidx = cidx_v[pl.ds(r0, L)]
            icidx = icidx_v[pl.ds(r0, L)]
            acc_l = jnp.zeros((L,), f32)
            acc_c = jnp.zeros((L,), f32)
            for d in range(DH):
                drow = jnp.full((L,), d, i32)
                lv = plsc.load_gather(ltab_v, [drow, lidx])
                ilv = plsc.load_gather(ltab_v, [drow, ilidx])
                acc_l = acc_l + jnp.abs(lv - ilv) * wlvec[d]
                cv = plsc.load_gather(ctab_v, [drow, cidx])
                icv = plsc.load_gather(ctab_v, [drow, icidx])
                acc_c = acc_c + jnp.abs(cv - icv) * wcvec[d]
            sig_l = 1.0 / (1.0 + jnp.exp(-(acc_l + blvec)))
            sig_c = 1.0 / (1.0 + jnp.exp(-(acc_c + bcvec)))
            gate_v[pl.ds(r0, L)] = sig_l * sig_c
            return ()

        lax.fori_loop(0, BPW // L, chunk_body, (), unroll=1)

        pltpu.sync_copy(uT_v, uT_out.at[:, sl])
        pltpu.sync_copy(iT_v, iT_out.at[:, sl])
        pltpu.sync_copy(gate_v, gate_out.at[sl])

    return sc_kernel(user, item, language, category, utab4, itab4, ltabT,
                     ctabT, item_languages, item_categories, wl, bl, wc, bc)


def _tc_dense(uT, iT, gate2d, W1uT, W1iT, b1c, W2T, b2c, w3c, b3):
    NB = 4096
    grid = (B // NB,)
    f32 = jnp.float32

    def body(uT_r, iT_r, gate_r, W1uT_r, W1iT_r, b1c_r, W2T_r, b2c_r,
             w3c_r, b3_r, out_r):
        h = jnp.dot(W1uT_r[...], uT_r[...], preferred_element_type=f32)
        h = h + jnp.dot(W1iT_r[...], iT_r[...], preferred_element_type=f32)
        h = jax.nn.relu(h + b1c_r[...])
        h = jax.nn.relu(jnp.dot(W2T_r[...], h, preferred_element_type=f32)
                        + b2c_r[...])
        base = jnp.sum(h * w3c_r[...], axis=0, keepdims=True) + b3_r[0, 0]
        out_r[...] = base * gate_r[...]

    colspec = lambda h: pl.BlockSpec((h, NB), lambda b: (0, b))
    full = lambda s: pl.BlockSpec(s, lambda b: (0,) * len(s))
    out = pl.pallas_call(
        body,
        grid=grid,
        in_specs=[
            colspec(D), colspec(D), colspec(1),
            full((128, D)), full((128, D)), full((128, 1)),
            full((64, 128)), full((64, 1)), full((64, 1)), full((1, 1)),
        ],
        out_specs=pl.BlockSpec((1, NB), lambda b: (0, b)),
        out_shape=jax.ShapeDtypeStruct((1, B), f32),
    )(uT, iT, gate2d, W1uT, W1iT, b1c, W2T, b2c, w3c, b3)
    return jnp.reshape(out, (B,))


def kernel(user, item, language, category, user_table, item_table,
           lang_table, cat_table, item_languages, item_categories,
           W_lang, b_lang, W_cat, b_cat, W1, b1, W2, b2, W3, b3):
    utab4, itab4 = _pack2(user_table.T, item_table.T)
    uT, iT, gate = _sc_gather(
        user, item, language, category, utab4, itab4,
        lang_table.T, cat_table.T, item_languages, item_categories,
        jnp.reshape(W_lang, (DH,)), jnp.broadcast_to(b_lang, (L,)),
        jnp.reshape(W_cat, (DH,)), jnp.broadcast_to(b_cat, (L,)))
    gate2d = jnp.reshape(gate, (1, B))
    W1uT = jnp.transpose(W1[:D])
    W1iT = jnp.transpose(W1[D:])
    b1c = jnp.reshape(b1, (128, 1))
    W2T = jnp.transpose(W2)
    b2c = jnp.reshape(b2, (64, 1))
    w3c = jnp.reshape(W3, (64, 1))
    b3c = jnp.reshape(b3, (1, 1))
    return _tc_dense(uT, iT, gate2d, W1uT, W1iT, b1c, W2T, b2c, w3c, b3c)


# speedup vs baseline: 1.6764x; 1.1971x over previous
import functools

import jax
import jax.numpy as jnp
from jax import lax
from jax.experimental import pallas as pl
from jax.experimental.pallas import tpu as pltpu
from jax.experimental.pallas import tpu_sc as plsc

B = 16384
D = 32
DH = D // 2
NL = 100
NCAT = 1000
NTAB = 1000000
NC = 2
NS = 16
NW = NC * NS
BPW = B // NW
L = 16
PACK_TG = 16
PACK_BC = PACK_TG * 8 * 128


def _bfpack_piece(piece, even_sel, odd_sel):
    f32 = jnp.float32
    i32 = jnp.int32
    ev = lax.dot_general(even_sel, piece, (((1,), (0,)), ((), ())),
                         preferred_element_type=f32)
    od = lax.dot_general(odd_sel, piece, (((1,), (0,)), ((), ())),
                         preferred_element_type=f32)
    lo = lax.bitcast_convert_type(ev.astype(jnp.bfloat16), jnp.int16)
    hi = lax.bitcast_convert_type(od.astype(jnp.bfloat16), jnp.int16)
    lo32 = lo.astype(i32) & jnp.int32(0xFFFF)
    hi32 = lax.shift_left(hi.astype(i32), 16)
    return lo32 | hi32


def _pack_body(xT_r, yT_r, esel_r, osel_r, outx_r, outy_r):
    x = xT_r[...]
    y = yT_r[...]
    es = esel_r[...]
    os_ = osel_r[...]
    for g in range(PACK_TG):
        sx = jnp.concatenate(
            [_bfpack_piece(x[:, (g * 8 + k) * 128:(g * 8 + k + 1) * 128],
                           es, os_) for k in range(8)], axis=0)
        outx_r[g] = jnp.transpose(sx)
        sy = jnp.concatenate(
            [_bfpack_piece(y[:, (g * 8 + k) * 128:(g * 8 + k + 1) * 128],
                           es, os_) for k in range(8)], axis=0)
        outy_r[g] = jnp.transpose(sy)


def _pack2(xT, yT):
    n = xT.shape[1]
    nblk = pl.cdiv(n, PACK_BC)
    dd = jnp.arange(D, dtype=jnp.int32)
    pp = jnp.arange(DH, dtype=jnp.int32)
    even_sel = (dd[None, :] == 2 * pp[:, None]).astype(jnp.float32)
    odd_sel = (dd[None, :] == 2 * pp[:, None] + 1).astype(jnp.float32)
    out_t = jax.ShapeDtypeStruct((nblk * PACK_TG, 128, 128), jnp.int32)
    outx, outy = pl.pallas_call(
        _pack_body,
        grid=(nblk,),
        in_specs=[pl.BlockSpec((D, PACK_BC), lambda c: (0, c)),
                  pl.BlockSpec((D, PACK_BC), lambda c: (0, c)),
                  pl.BlockSpec((DH, D), lambda c: (0, 0)),
                  pl.BlockSpec((DH, D), lambda c: (0, 0))],
        out_specs=[pl.BlockSpec((PACK_TG, 128, 128), lambda c: (c, 0, 0)),
                   pl.BlockSpec((PACK_TG, 128, 128), lambda c: (c, 0, 0))],
        out_shape=[out_t, out_t],
    )(xT, yT, even_sel, odd_sel)
    m = nblk * PACK_TG * 128
    return jnp.reshape(outx, (m, 128)), jnp.reshape(outy, (m, 128))


def _sc_gather(user, item, language, category, utab4, itab4, ltabT, ctabT,
               item_languages, item_categories, wl, bl, wc, bc):
    f32 = jnp.float32
    i32 = jnp.int32
    mesh = plsc.VectorSubcoreMesh(core_axis_name="c", subcore_axis_name="s")

    @functools.partial(
        pl.kernel,
        out_type=[
            jax.ShapeDtypeStruct((D, B), f32),
            jax.ShapeDtypeStruct((D, B), f32),
            jax.ShapeDtypeStruct((B,), f32),
        ],
        mesh=mesh,
        compiler_params=pltpu.CompilerParams(use_tc_tiling_on_sc=True,
                                             needs_layout_passes=False),
        scratch_types=[
            pltpu.VMEM((BPW,), i32),
            pltpu.VMEM((BPW,), i32),
            pltpu.VMEM((BPW,), i32),
            pltpu.VMEM((BPW,), i32),
            pltpu.VMEM((BPW,), i32),
            pltpu.VMEM((BPW,), i32),
            pltpu.VMEM((BPW,), i32),
            pltpu.VMEM((BPW,), i32),
            pltpu.VMEM((BPW, 128), i32),
            pltpu.VMEM((D, BPW), f32),
            pltpu.VMEM((D, BPW), f32),
            pltpu.VMEM((DH, NL), f32),
            pltpu.VMEM((DH, NCAT), f32),
            pltpu.VMEM((DH,), f32),
            pltpu.VMEM((DH,), f32),
            pltpu.VMEM((L,), f32),
            pltpu.VMEM((L,), f32),
            pltpu.VMEM((BPW,), f32),
            pltpu.SemaphoreType.DMA,
            pltpu.SemaphoreType.DMA,
        ],
    )
    def sc_kernel(user_h, item_h, lang_h, cat_h, utab4_h, itab4_h, ltabT_h,
                  ctabT_h, ilang_h, icat_h, wl_h, bl_h, wc_h, bc_h,
                  uT_out, iT_out, gate_out,
                  uidx_v, iidx_v, lidx_v, cidx_v, ilidx_v, icidx_v,
                  uq_v, iq_v, x128_v, uT_v, iT_v, ltab_v, ctab_v,
                  wl_v, wc_v, bl_v, bc_v, gate_v, sem, sem2):
        wid = lax.axis_index("s") * NC + lax.axis_index("c")
        base = wid * BPW
        sl = pl.ds(base, BPW)
        pltpu.sync_copy(user_h.at[sl], uidx_v)
        pltpu.sync_copy(item_h.at[sl], iidx_v)
        pltpu.sync_copy(lang_h.at[sl], lidx_v)
        pltpu.sync_copy(cat_h.at[sl], cidx_v)
        m1 = pltpu.async_copy(ilang_h.at[iidx_v], ilidx_v, sem2)
        m2 = pltpu.async_copy(icat_h.at[iidx_v], icidx_v, sem2)
        pltpu.sync_copy(ltabT_h, ltab_v)
        pltpu.sync_copy(ctabT_h, ctab_v)
        pltpu.sync_copy(wl_h, wl_v)
        pltpu.sync_copy(wc_h, wc_v)
        pltpu.sync_copy(bl_h, bl_v)
        pltpu.sync_copy(bc_h, bc_v)

        def qbody(ci, _):
            s = pl.ds(ci * L, L)
            u = uidx_v[s]
            i = iidx_v[s]
            uq_v[s] = lax.shift_left(lax.shift_right_logical(u, 10), 7) \
                + (u & 127)
            iq_v[s] = lax.shift_left(lax.shift_right_logical(i, 10), 7) \
                + (i & 127)
            return ()

        lax.fori_loop(0, BPW // L, qbody, (), unroll=4)

        lane = lax.iota(i32, L)

        himask = jnp.int32(-65536)

        def extract(idx_ref, dst_ref):
            def ebody(ci, _):
                r0 = ci * L
                rows = r0 + lane
                basecol = (lax.shift_right_logical(idx_ref[pl.ds(r0, L)], 7)
                           & 7) * L
                for p in range(DH):
                    v = plsc.load_gather(x128_v, [rows, basecol + p])
                    dst_ref[2 * p, pl.ds(r0, L)] = \
                        plsc.bitcast(lax.shift_left(v, 16), f32)
                    dst_ref[2 * p + 1, pl.ds(r0, L)] = \
                        plsc.bitcast(v & himask, f32)
                return ()

            lax.fori_loop(0, BPW // L, ebody, (), unroll=1)

        pltpu.async_copy(utab4_h.at[uq_v], x128_v, sem).wait()
        extract(uidx_v, uT_v)
        pltpu.async_copy(itab4_h.at[iq_v], x128_v, sem).wait()
        extract(iidx_v, iT_v)

        m1.wait()
        m2.wait()

        wlvec = wl_v[...]
        wcvec = wc_v[...]
        blvec = bl_v[...]
        bcvec = bc_v[...]

        def chunk_body(ci, _):
            r0 = ci * L
            lidx = lidx_v[pl.ds(r0, L)]
            ilidx = ilidx_v[pl.ds(r0, L)]
            cidx = cidx_v[pl.ds(r0, L)]
            icidx = icidx_v[pl.ds(r0, L)]
            acc_l = jnp.zeros((L,), f32)
            acc_c = jnp.zeros((L,), f32)
            for d in range(DH):
                drow = jnp.full((L,), d, i32)
                lv = plsc.load_gather(ltab_v, [drow, lidx])
                ilv = plsc.load_gather(ltab_v, [drow, ilidx])
                acc_l = acc_l + jnp.abs(lv - ilv) * wlvec[d]
                cv = plsc.load_gather(ctab_v, [drow, cidx])
                icv = plsc.load_gather(ctab_v, [drow, icidx])
                acc_c = acc_c + jnp.abs(cv - icv) * wcvec[d]
            sig_l = 1.0 / (1.0 + jnp.exp(-(acc_l + blvec)))
            sig_c = 1.0 / (1.0 + jnp.exp(-(acc_c + bcvec)))
            gate_v[pl.ds(r0, L)] = sig_l * sig_c
            return ()

        lax.fori_loop(0, BPW // L, chunk_body, (), unroll=1)

        pltpu.sync_copy(uT_v, uT_out.at[:, sl])
        pltpu.sync_copy(iT_v, iT_out.at[:, sl])
        pltpu.sync_copy(gate_v, gate_out.at[sl])

    return sc_kernel(user, item, language, category, utab4, itab4, ltabT,
                     ctabT, item_languages, item_categories, wl, bl, wc, bc)


def _tc_dense(uT, iT, gate2d, W1uT, W1iT, b1c, W2T, b2c, w3c, b3):
    NB = 4096
    grid = (B // NB,)
    f32 = jnp.float32

    def body(uT_r, iT_r, gate_r, W1uT_r, W1iT_r, b1c_r, W2T_r, b2c_r,
             w3c_r, b3_r, out_r):
        h = jnp.dot(W1uT_r[...], uT_r[...], preferred_element_type=f32)
        h = h + jnp.dot(W1iT_r[...], iT_r[...], preferred_element_type=f32)
        h = jax.nn.relu(h + b1c_r[...])
        h = jax.nn.relu(jnp.dot(W2T_r[...], h, preferred_element_type=f32)
                        + b2c_r[...])
        base = jnp.sum(h * w3c_r[...], axis=0, keepdims=True) + b3_r[0, 0]
        out_r[...] = base * gate_r[...]

    colspec = lambda h: pl.BlockSpec((h, NB), lambda b: (0, b))
    full = lambda s: pl.BlockSpec(s, lambda b: (0,) * len(s))
    out = pl.pallas_call(
        body,
        grid=grid,
        in_specs=[
            colspec(D), colspec(D), colspec(1),
            full((128, D)), full((128, D)), full((128, 1)),
            full((64, 128)), full((64, 1)), full((64, 1)), full((1, 1)),
        ],
        out_specs=pl.BlockSpec((1, NB), lambda b: (0, b)),
        out_shape=jax.ShapeDtypeStruct((1, B), f32),
    )(uT, iT, gate2d, W1uT, W1iT, b1c, W2T, b2c, w3c, b3)
    return jnp.reshape(out, (B,))


def kernel(user, item, language, category, user_table, item_table,
           lang_table, cat_table, item_languages, item_categories,
           W_lang, b_lang, W_cat, b_cat, W1, b1, W2, b2, W3, b3):
    utab4, itab4 = _pack2(user_table.T, item_table.T)
    uT, iT, gate = _sc_gather(
        user, item, language, category, utab4, itab4,
        lang_table.T, cat_table.T, item_languages, item_categories,
        jnp.reshape(W_lang, (DH,)), jnp.broadcast_to(b_lang, (L,)),
        jnp.reshape(W_cat, (DH,)), jnp.broadcast_to(b_cat, (L,)))
    gate2d = jnp.reshape(gate, (1, B))
    W1uT = jnp.transpose(W1[:D])
    W1iT = jnp.transpose(W1[D:])
    b1c = jnp.reshape(b1, (128, 1))
    W2T = jnp.transpose(W2)
    b2c = jnp.reshape(b2, (64, 1))
    w3c = jnp.reshape(W3, (64, 1))
    b3c = jnp.reshape(b3, (1, 1))
    return _tc_dense(uT, iT, gate2d, W1uT, W1iT, b1c, W2T, b2c, w3c, b3c)


# speedup vs baseline: 1.7465x; 1.0418x over previous
import functools

import jax
import jax.numpy as jnp
from jax import lax
from jax.experimental import pallas as pl
from jax.experimental.pallas import tpu as pltpu
from jax.experimental.pallas import tpu_sc as plsc

B = 16384
D = 32
DH = D // 2
NL = 100
NCAT = 1000
NTAB = 1000000
NC = 2
NS = 16
NW = NC * NS
BPW = B // NW
L = 16
PACK_TG = 32
PACK_BC = PACK_TG * 8 * 128


def _bfpack_piece(piece, even_sel, odd_sel):
    f32 = jnp.float32
    i32 = jnp.int32
    ev = lax.dot_general(even_sel, piece, (((1,), (0,)), ((), ())),
                         preferred_element_type=f32)
    od = lax.dot_general(odd_sel, piece, (((1,), (0,)), ((), ())),
                         preferred_element_type=f32)
    lo = lax.bitcast_convert_type(ev.astype(jnp.bfloat16), jnp.int16)
    hi = lax.bitcast_convert_type(od.astype(jnp.bfloat16), jnp.int16)
    lo32 = lo.astype(i32) & jnp.int32(0xFFFF)
    hi32 = lax.shift_left(hi.astype(i32), 16)
    return lo32 | hi32


def _pack_body(xT_r, yT_r, esel_r, osel_r, outx_r, outy_r):
    x = xT_r[...]
    y = yT_r[...]
    es = esel_r[...]
    os_ = osel_r[...]
    for g in range(PACK_TG):
        sx = jnp.concatenate(
            [_bfpack_piece(x[:, (g * 8 + k) * 128:(g * 8 + k + 1) * 128],
                           es, os_) for k in range(8)], axis=0)
        outx_r[g] = jnp.transpose(sx)
        sy = jnp.concatenate(
            [_bfpack_piece(y[:, (g * 8 + k) * 128:(g * 8 + k + 1) * 128],
                           es, os_) for k in range(8)], axis=0)
        outy_r[g] = jnp.transpose(sy)


def _pack2(xT, yT):
    n = xT.shape[1]
    nblk = pl.cdiv(n, PACK_BC)
    dd = jnp.arange(D, dtype=jnp.int32)
    pp = jnp.arange(DH, dtype=jnp.int32)
    even_sel = (dd[None, :] == 2 * pp[:, None]).astype(jnp.float32)
    odd_sel = (dd[None, :] == 2 * pp[:, None] + 1).astype(jnp.float32)
    out_t = jax.ShapeDtypeStruct((nblk * PACK_TG, 128, 128), jnp.int32)
    outx, outy = pl.pallas_call(
        _pack_body,
        grid=(nblk,),
        in_specs=[pl.BlockSpec((D, PACK_BC), lambda c: (0, c)),
                  pl.BlockSpec((D, PACK_BC), lambda c: (0, c)),
                  pl.BlockSpec((DH, D), lambda c: (0, 0)),
                  pl.BlockSpec((DH, D), lambda c: (0, 0))],
        out_specs=[pl.BlockSpec((PACK_TG, 128, 128), lambda c: (c, 0, 0)),
                   pl.BlockSpec((PACK_TG, 128, 128), lambda c: (c, 0, 0))],
        out_shape=[out_t, out_t],
    )(xT, yT, even_sel, odd_sel)
    m = nblk * PACK_TG * 128
    return jnp.reshape(outx, (m, 128)), jnp.reshape(outy, (m, 128))


def _sc_gather(user, item, language, category, utab4, itab4, ltabT, ctabT,
               item_languages, item_categories, wl, bl, wc, bc):
    f32 = jnp.float32
    i32 = jnp.int32
    mesh = plsc.VectorSubcoreMesh(core_axis_name="c", subcore_axis_name="s")

    @functools.partial(
        pl.kernel,
        out_type=[
            jax.ShapeDtypeStruct((D, B), f32),
            jax.ShapeDtypeStruct((D, B), f32),
            jax.ShapeDtypeStruct((B,), f32),
        ],
        mesh=mesh,
        compiler_params=pltpu.CompilerParams(use_tc_tiling_on_sc=True,
                                             needs_layout_passes=False),
        scratch_types=[
            pltpu.VMEM((BPW,), i32),
            pltpu.VMEM((BPW,), i32),
            pltpu.VMEM((BPW,), i32),
            pltpu.VMEM((BPW,), i32),
            pltpu.VMEM((BPW,), i32),
            pltpu.VMEM((BPW,), i32),
            pltpu.VMEM((BPW,), i32),
            pltpu.VMEM((BPW,), i32),
            pltpu.VMEM((BPW, 128), i32),
            pltpu.VMEM((D, BPW), f32),
            pltpu.VMEM((D, BPW), f32),
            pltpu.VMEM((DH, NL), f32),
            pltpu.VMEM((DH, NCAT), f32),
            pltpu.VMEM((DH,), f32),
            pltpu.VMEM((DH,), f32),
            pltpu.VMEM((L,), f32),
            pltpu.VMEM((L,), f32),
            pltpu.VMEM((BPW,), f32),
            pltpu.SemaphoreType.DMA,
            pltpu.SemaphoreType.DMA,
        ],
    )
    def sc_kernel(user_h, item_h, lang_h, cat_h, utab4_h, itab4_h, ltabT_h,
                  ctabT_h, ilang_h, icat_h, wl_h, bl_h, wc_h, bc_h,
                  uT_out, iT_out, gate_out,
                  uidx_v, iidx_v, lidx_v, cidx_v, ilidx_v, icidx_v,
                  uq_v, iq_v, x128_v, uT_v, iT_v, ltab_v, ctab_v,
                  wl_v, wc_v, bl_v, bc_v, gate_v, sem, sem2):
        wid = lax.axis_index("s") * NC + lax.axis_index("c")
        base = wid * BPW
        sl = pl.ds(base, BPW)
        pltpu.sync_copy(user_h.at[sl], uidx_v)
        pltpu.sync_copy(item_h.at[sl], iidx_v)
        pltpu.sync_copy(lang_h.at[sl], lidx_v)
        pltpu.sync_copy(cat_h.at[sl], cidx_v)
        m1 = pltpu.async_copy(ilang_h.at[iidx_v], ilidx_v, sem2)
        m2 = pltpu.async_copy(icat_h.at[iidx_v], icidx_v, sem2)
        pltpu.sync_copy(ltabT_h, ltab_v)
        pltpu.sync_copy(ctabT_h, ctab_v)
        pltpu.sync_copy(wl_h, wl_v)
        pltpu.sync_copy(wc_h, wc_v)
        pltpu.sync_copy(bl_h, bl_v)
        pltpu.sync_copy(bc_h, bc_v)

        def qbody(ci, _):
            s = pl.ds(ci * L, L)
            u = uidx_v[s]
            i = iidx_v[s]
            uq_v[s] = lax.shift_left(lax.shift_right_logical(u, 10), 7) \
                + (u & 127)
            iq_v[s] = lax.shift_left(lax.shift_right_logical(i, 10), 7) \
                + (i & 127)
            return ()

        lax.fori_loop(0, BPW // L, qbody, (), unroll=4)

        lane = lax.iota(i32, L)

        himask = jnp.int32(-65536)

        def extract(idx_ref, dst_ref):
            def ebody(ci, _):
                r0 = ci * L
                rows = r0 + lane
                basecol = (lax.shift_right_logical(idx_ref[pl.ds(r0, L)], 7)
                           & 7) * L
                for p in range(DH):
                    v = plsc.load_gather(x128_v, [rows, basecol + p])
                    dst_ref[2 * p, pl.ds(r0, L)] = \
                        plsc.bitcast(lax.shift_left(v, 16), f32)
                    dst_ref[2 * p + 1, pl.ds(r0, L)] = \
                        plsc.bitcast(v & himask, f32)
                return ()

            lax.fori_loop(0, BPW // L, ebody, (), unroll=1)

        pltpu.async_copy(utab4_h.at[uq_v], x128_v, sem).wait()
        extract(uidx_v, uT_v)
        pltpu.async_copy(itab4_h.at[iq_v], x128_v, sem).wait()
        extract(iidx_v, iT_v)

        m1.wait()
        m2.wait()

        wlvec = wl_v[...]
        wcvec = wc_v[...]
        blvec = bl_v[...]
        bcvec = bc_v[...]

        def chunk_body(ci, _):
            r0 = ci * L
            lidx = lidx_v[pl.ds(r0, L)]
            ilidx = ilidx_v[pl.ds(r0, L)]
            cidx = cidx_v[pl.ds(r0, L)]
            icidx = icidx_v[pl.ds(r0, L)]
            acc_l = jnp.zeros((L,), f32)
            acc_c = jnp.zeros((L,), f32)
            for d in range(DH):
                drow = jnp.full((L,), d, i32)
                lv = plsc.load_gather(ltab_v, [drow, lidx])
                ilv = plsc.load_gather(ltab_v, [drow, ilidx])
                acc_l = acc_l + jnp.abs(lv - ilv) * wlvec[d]
                cv = plsc.load_gather(ctab_v, [drow, cidx])
                icv = plsc.load_gather(ctab_v, [drow, icidx])
                acc_c = acc_c + jnp.abs(cv - icv) * wcvec[d]
            sig_l = 1.0 / (1.0 + jnp.exp(-(acc_l + blvec)))
            sig_c = 1.0 / (1.0 + jnp.exp(-(acc_c + bcvec)))
            gate_v[pl.ds(r0, L)] = sig_l * sig_c
            return ()

        lax.fori_loop(0, BPW // L, chunk_body, (), unroll=1)

        pltpu.sync_copy(uT_v, uT_out.at[:, sl])
        pltpu.sync_copy(iT_v, iT_out.at[:, sl])
        pltpu.sync_copy(gate_v, gate_out.at[sl])

    return sc_kernel(user, item, language, category, utab4, itab4, ltabT,
                     ctabT, item_languages, item_categories, wl, bl, wc, bc)


def _tc_dense(uT, iT, gate2d, W1uT, W1iT, b1c, W2T, b2c, w3c, b3):
    NB = 4096
    grid = (B // NB,)
    f32 = jnp.float32

    def body(uT_r, iT_r, gate_r, W1uT_r, W1iT_r, b1c_r, W2T_r, b2c_r,
             w3c_r, b3_r, out_r):
        h = jnp.dot(W1uT_r[...], uT_r[...], preferred_element_type=f32)
        h = h + jnp.dot(W1iT_r[...], iT_r[...], preferred_element_type=f32)
        h = jax.nn.relu(h + b1c_r[...])
        h = jax.nn.relu(jnp.dot(W2T_r[...], h, preferred_element_type=f32)
                        + b2c_r[...])
        base = jnp.sum(h * w3c_r[...], axis=0, keepdims=True) + b3_r[0, 0]
        out_r[...] = base * gate_r[...]

    colspec = lambda h: pl.BlockSpec((h, NB), lambda b: (0, b))
    full = lambda s: pl.BlockSpec(s, lambda b: (0,) * len(s))
    out = pl.pallas_call(
        body,
        grid=grid,
        in_specs=[
            colspec(D), colspec(D), colspec(1),
            full((128, D)), full((128, D)), full((128, 1)),
            full((64, 128)), full((64, 1)), full((64, 1)), full((1, 1)),
        ],
        out_specs=pl.BlockSpec((1, NB), lambda b: (0, b)),
        out_shape=jax.ShapeDtypeStruct((1, B), f32),
    )(uT, iT, gate2d, W1uT, W1iT, b1c, W2T, b2c, w3c, b3)
    return jnp.reshape(out, (B,))


def kernel(user, item, language, category, user_table, item_table,
           lang_table, cat_table, item_languages, item_categories,
           W_lang, b_lang, W_cat, b_cat, W1, b1, W2, b2, W3, b3):
    utab4, itab4 = _pack2(user_table.T, item_table.T)
    uT, iT, gate = _sc_gather(
        user, item, language, category, utab4, itab4,
        lang_table.T, cat_table.T, item_languages, item_categories,
        jnp.reshape(W_lang, (DH,)), jnp.broadcast_to(b_lang, (L,)),
        jnp.reshape(W_cat, (DH,)), jnp.broadcast_to(b_cat, (L,)))
    gate2d = jnp.reshape(gate, (1, B))
    W1uT = jnp.transpose(W1[:D])
    W1iT = jnp.transpose(W1[D:])
    b1c = jnp.reshape(b1, (128, 1))
    W2T = jnp.transpose(W2)
    b2c = jnp.reshape(b2, (64, 1))
    w3c = jnp.reshape(W3, (64, 1))
    b3c = jnp.reshape(b3, (1, 1))
    return _tc_dense(uT, iT, gate2d, W1uT, W1iT, b1c, W2T, b2c, w3c, b3c)


# speedup vs baseline: 1.7616x; 1.0086x over previous
import functools

import jax
import jax.numpy as jnp
from jax import lax
from jax.experimental import pallas as pl
from jax.experimental.pallas import tpu as pltpu
from jax.experimental.pallas import tpu_sc as plsc

B = 16384
D = 32
DH = D // 2
NL = 100
NCAT = 1000
NTAB = 1000000
NC = 2
NS = 16
NW = NC * NS
BPW = B // NW
L = 16
PACK_TG = 61
PACK_BC = PACK_TG * 8 * 128


def _bfpack_piece(piece, even_sel, odd_sel):
    f32 = jnp.float32
    i32 = jnp.int32
    ev = lax.dot_general(even_sel, piece, (((1,), (0,)), ((), ())),
                         preferred_element_type=f32)
    od = lax.dot_general(odd_sel, piece, (((1,), (0,)), ((), ())),
                         preferred_element_type=f32)
    lo = lax.bitcast_convert_type(ev.astype(jnp.bfloat16), jnp.int16)
    hi = lax.bitcast_convert_type(od.astype(jnp.bfloat16), jnp.int16)
    lo32 = lo.astype(i32) & jnp.int32(0xFFFF)
    hi32 = lax.shift_left(hi.astype(i32), 16)
    return lo32 | hi32


def _pack_body(xT_r, yT_r, esel_r, osel_r, outx_r, outy_r):
    x = xT_r[...]
    y = yT_r[...]
    es = esel_r[...]
    os_ = osel_r[...]
    for g in range(PACK_TG):
        sx = jnp.concatenate(
            [_bfpack_piece(x[:, (g * 8 + k) * 128:(g * 8 + k + 1) * 128],
                           es, os_) for k in range(8)], axis=0)
        outx_r[g] = jnp.transpose(sx)
        sy = jnp.concatenate(
            [_bfpack_piece(y[:, (g * 8 + k) * 128:(g * 8 + k + 1) * 128],
                           es, os_) for k in range(8)], axis=0)
        outy_r[g] = jnp.transpose(sy)


def _pack2(xT, yT):
    n = xT.shape[1]
    nblk = pl.cdiv(n, PACK_BC)
    dd = jnp.arange(D, dtype=jnp.int32)
    pp = jnp.arange(DH, dtype=jnp.int32)
    even_sel = (dd[None, :] == 2 * pp[:, None]).astype(jnp.float32)
    odd_sel = (dd[None, :] == 2 * pp[:, None] + 1).astype(jnp.float32)
    out_t = jax.ShapeDtypeStruct((nblk * PACK_TG, 128, 128), jnp.int32)
    outx, outy = pl.pallas_call(
        _pack_body,
        grid=(nblk,),
        in_specs=[pl.BlockSpec((D, PACK_BC), lambda c: (0, c)),
                  pl.BlockSpec((D, PACK_BC), lambda c: (0, c)),
                  pl.BlockSpec((DH, D), lambda c: (0, 0)),
                  pl.BlockSpec((DH, D), lambda c: (0, 0))],
        out_specs=[pl.BlockSpec((PACK_TG, 128, 128), lambda c: (c, 0, 0)),
                   pl.BlockSpec((PACK_TG, 128, 128), lambda c: (c, 0, 0))],
        out_shape=[out_t, out_t],
    )(xT, yT, even_sel, odd_sel)
    m = nblk * PACK_TG * 128
    return jnp.reshape(outx, (m, 128)), jnp.reshape(outy, (m, 128))


def _sc_gather(user, item, language, category, utab4, itab4, ltabT, ctabT,
               item_languages, item_categories, wl, bl, wc, bc):
    f32 = jnp.float32
    i32 = jnp.int32
    mesh = plsc.VectorSubcoreMesh(core_axis_name="c", subcore_axis_name="s")

    @functools.partial(
        pl.kernel,
        out_type=[
            jax.ShapeDtypeStruct((D, B), f32),
            jax.ShapeDtypeStruct((D, B), f32),
            jax.ShapeDtypeStruct((B,), f32),
        ],
        mesh=mesh,
        compiler_params=pltpu.CompilerParams(use_tc_tiling_on_sc=True,
                                             needs_layout_passes=False),
        scratch_types=[
            pltpu.VMEM((BPW,), i32),
            pltpu.VMEM((BPW,), i32),
            pltpu.VMEM((BPW,), i32),
            pltpu.VMEM((BPW,), i32),
            pltpu.VMEM((BPW,), i32),
            pltpu.VMEM((BPW,), i32),
            pltpu.VMEM((BPW,), i32),
            pltpu.VMEM((BPW,), i32),
            pltpu.VMEM((BPW, 128), i32),
            pltpu.VMEM((D, BPW), f32),
            pltpu.VMEM((D, BPW), f32),
            pltpu.VMEM((DH, NL), f32),
            pltpu.VMEM((DH, NCAT), f32),
            pltpu.VMEM((DH,), f32),
            pltpu.VMEM((DH,), f32),
            pltpu.VMEM((L,), f32),
            pltpu.VMEM((L,), f32),
            pltpu.VMEM((BPW,), f32),
            pltpu.SemaphoreType.DMA,
            pltpu.SemaphoreType.DMA,
        ],
    )
    def sc_kernel(user_h, item_h, lang_h, cat_h, utab4_h, itab4_h, ltabT_h,
                  ctabT_h, ilang_h, icat_h, wl_h, bl_h, wc_h, bc_h,
                  uT_out, iT_out, gate_out,
                  uidx_v, iidx_v, lidx_v, cidx_v, ilidx_v, icidx_v,
                  uq_v, iq_v, x128_v, uT_v, iT_v, ltab_v, ctab_v,
                  wl_v, wc_v, bl_v, bc_v, gate_v, sem, sem2):
        wid = lax.axis_index("s") * NC + lax.axis_index("c")
        base = wid * BPW
        sl = pl.ds(base, BPW)
        pltpu.sync_copy(user_h.at[sl], uidx_v)
        pltpu.sync_copy(item_h.at[sl], iidx_v)
        pltpu.sync_copy(lang_h.at[sl], lidx_v)
        pltpu.sync_copy(cat_h.at[sl], cidx_v)
        m1 = pltpu.async_copy(ilang_h.at[iidx_v], ilidx_v, sem2)
        m2 = pltpu.async_copy(icat_h.at[iidx_v], icidx_v, sem2)
        pltpu.sync_copy(ltabT_h, ltab_v)
        pltpu.sync_copy(ctabT_h, ctab_v)
        pltpu.sync_copy(wl_h, wl_v)
        pltpu.sync_copy(wc_h, wc_v)
        pltpu.sync_copy(bl_h, bl_v)
        pltpu.sync_copy(bc_h, bc_v)

        def qbody(ci, _):
            s = pl.ds(ci * L, L)
            u = uidx_v[s]
            i = iidx_v[s]
            uq_v[s] = lax.shift_left(lax.shift_right_logical(u, 10), 7) \
                + (u & 127)
            iq_v[s] = lax.shift_left(lax.shift_right_logical(i, 10), 7) \
                + (i & 127)
            return ()

        lax.fori_loop(0, BPW // L, qbody, (), unroll=4)

        lane = lax.iota(i32, L)

        himask = jnp.int32(-65536)

        def extract(idx_ref, dst_ref):
            def ebody(ci, _):
                r0 = ci * L
                rows = r0 + lane
                basecol = (lax.shift_right_logical(idx_ref[pl.ds(r0, L)], 7)
                           & 7) * L
                for p in range(DH):
                    v = plsc.load_gather(x128_v, [rows, basecol + p])
                    dst_ref[2 * p, pl.ds(r0, L)] = \
                        plsc.bitcast(lax.shift_left(v, 16), f32)
                    dst_ref[2 * p + 1, pl.ds(r0, L)] = \
                        plsc.bitcast(v & himask, f32)
                return ()

            lax.fori_loop(0, BPW // L, ebody, (), unroll=1)

        pltpu.async_copy(utab4_h.at[uq_v], x128_v, sem).wait()
        extract(uidx_v, uT_v)
        pltpu.async_copy(itab4_h.at[iq_v], x128_v, sem).wait()
        extract(iidx_v, iT_v)

        m1.wait()
        m2.wait()

        wlvec = wl_v[...]
        wcvec = wc_v[...]
        blvec = bl_v[...]
        bcvec = bc_v[...]

        def chunk_body(ci, _):
            r0 = ci * L
            lidx = lidx_v[pl.ds(r0, L)]
            ilidx = ilidx_v[pl.ds(r0, L)]
            cidx = cidx_v[pl.ds(r0, L)]
            icidx = icidx_v[pl.ds(r0, L)]
            acc_l = jnp.zeros((L,), f32)
            acc_c = jnp.zeros((L,), f32)
            for d in range(DH):
                drow = jnp.full((L,), d, i32)
                lv = plsc.load_gather(ltab_v, [drow, lidx])
                ilv = plsc.load_gather(ltab_v, [drow, ilidx])
                acc_l = acc_l + jnp.abs(lv - ilv) * wlvec[d]
                cv = plsc.load_gather(ctab_v, [drow, cidx])
                icv = plsc.load_gather(ctab_v, [drow, icidx])
                acc_c = acc_c + jnp.abs(cv - icv) * wcvec[d]
            sig_l = 1.0 / (1.0 + jnp.exp(-(acc_l + blvec)))
            sig_c = 1.0 / (1.0 + jnp.exp(-(acc_c + bcvec)))
            gate_v[pl.ds(r0, L)] = sig_l * sig_c
            return ()

        lax.fori_loop(0, BPW // L, chunk_body, (), unroll=1)

        pltpu.sync_copy(uT_v, uT_out.at[:, sl])
        pltpu.sync_copy(iT_v, iT_out.at[:, sl])
        pltpu.sync_copy(gate_v, gate_out.at[sl])

    return sc_kernel(user, item, language, category, utab4, itab4, ltabT,
                     ctabT, item_languages, item_categories, wl, bl, wc, bc)


def _tc_dense(uT, iT, gate2d, W1uT, W1iT, b1c, W2T, b2c, w3c, b3):
    NB = 4096
    grid = (B // NB,)
    f32 = jnp.float32

    def body(uT_r, iT_r, gate_r, W1uT_r, W1iT_r, b1c_r, W2T_r, b2c_r,
             w3c_r, b3_r, out_r):
        h = jnp.dot(W1uT_r[...], uT_r[...], preferred_element_type=f32)
        h = h + jnp.dot(W1iT_r[...], iT_r[...], preferred_element_type=f32)
        h = jax.nn.relu(h + b1c_r[...])
        h = jax.nn.relu(jnp.dot(W2T_r[...], h, preferred_element_type=f32)
                        + b2c_r[...])
        base = jnp.sum(h * w3c_r[...], axis=0, keepdims=True) + b3_r[0, 0]
        out_r[...] = base * gate_r[...]

    colspec = lambda h: pl.BlockSpec((h, NB), lambda b: (0, b))
    full = lambda s: pl.BlockSpec(s, lambda b: (0,) * len(s))
    out = pl.pallas_call(
        body,
        grid=grid,
        in_specs=[
            colspec(D), colspec(D), colspec(1),
            full((128, D)), full((128, D)), full((128, 1)),
            full((64, 128)), full((64, 1)), full((64, 1)), full((1, 1)),
        ],
        out_specs=pl.BlockSpec((1, NB), lambda b: (0, b)),
        out_shape=jax.ShapeDtypeStruct((1, B), f32),
    )(uT, iT, gate2d, W1uT, W1iT, b1c, W2T, b2c, w3c, b3)
    return jnp.reshape(out, (B,))


def kernel(user, item, language, category, user_table, item_table,
           lang_table, cat_table, item_languages, item_categories,
           W_lang, b_lang, W_cat, b_cat, W1, b1, W2, b2, W3, b3):
    utab4, itab4 = _pack2(user_table.T, item_table.T)
    uT, iT, gate = _sc_gather(
        user, item, language, category, utab4, itab4,
        lang_table.T, cat_table.T, item_languages, item_categories,
        jnp.reshape(W_lang, (DH,)), jnp.broadcast_to(b_lang, (L,)),
        jnp.reshape(W_cat, (DH,)), jnp.broadcast_to(b_cat, (L,)))
    gate2d = jnp.reshape(gate, (1, B))
    W1uT = jnp.transpose(W1[:D])
    W1iT = jnp.transpose(W1[D:])
    b1c = jnp.reshape(b1, (128, 1))
    W2T = jnp.transpose(W2)
    b2c = jnp.reshape(b2, (64, 1))
    w3c = jnp.reshape(W3, (64, 1))
    b3c = jnp.reshape(b3, (1, 1))
    return _tc_dense(uT, iT, gate2d, W1uT, W1iT, b1c, W2T, b2c, w3c, b3c)


# speedup vs baseline: 1.7670x; 1.0031x over previous
import functools

import jax
import jax.numpy as jnp
from jax import lax
from jax.experimental import pallas as pl
from jax.experimental.pallas import tpu as pltpu
from jax.experimental.pallas import tpu_sc as plsc

B = 16384
D = 32
DH = D // 2
NL = 100
NCAT = 1000
NTAB = 1000000
NC = 2
NS = 16
NW = NC * NS
BPW = B // NW
L = 16
PACK_TG = 61
PACK_BC = PACK_TG * 8 * 128


def _bfpack_piece(piece, even_sel, odd_sel):
    f32 = jnp.float32
    i32 = jnp.int32
    ev = lax.dot_general(even_sel, piece, (((1,), (0,)), ((), ())),
                         preferred_element_type=f32)
    od = lax.dot_general(odd_sel, piece, (((1,), (0,)), ((), ())),
                         preferred_element_type=f32)
    lo = lax.bitcast_convert_type(ev.astype(jnp.bfloat16), jnp.int16)
    hi = lax.bitcast_convert_type(od.astype(jnp.bfloat16), jnp.int16)
    lo32 = lo.astype(i32) & jnp.int32(0xFFFF)
    hi32 = lax.shift_left(hi.astype(i32), 16)
    return lo32 | hi32


def _pack_body(xT_r, yT_r, esel_r, osel_r, outx_r, outy_r):
    x = xT_r[...]
    y = yT_r[...]
    es = esel_r[...]
    os_ = osel_r[...]
    for g in range(PACK_TG):
        sx = jnp.concatenate(
            [_bfpack_piece(x[:, (g * 8 + k) * 128:(g * 8 + k + 1) * 128],
                           es, os_) for k in range(8)], axis=0)
        outx_r[g] = jnp.transpose(sx)
        sy = jnp.concatenate(
            [_bfpack_piece(y[:, (g * 8 + k) * 128:(g * 8 + k + 1) * 128],
                           es, os_) for k in range(8)], axis=0)
        outy_r[g] = jnp.transpose(sy)


def _pack2(xT, yT):
    n = xT.shape[1]
    nblk = pl.cdiv(n, PACK_BC)
    dd = jnp.arange(D, dtype=jnp.int32)
    pp = jnp.arange(DH, dtype=jnp.int32)
    even_sel = (dd[None, :] == 2 * pp[:, None]).astype(jnp.float32)
    odd_sel = (dd[None, :] == 2 * pp[:, None] + 1).astype(jnp.float32)
    out_t = jax.ShapeDtypeStruct((nblk * PACK_TG, 128, 128), jnp.int32)
    outx, outy = pl.pallas_call(
        _pack_body,
        grid=(nblk,),
        in_specs=[pl.BlockSpec((D, PACK_BC), lambda c: (0, c)),
                  pl.BlockSpec((D, PACK_BC), lambda c: (0, c)),
                  pl.BlockSpec((DH, D), lambda c: (0, 0)),
                  pl.BlockSpec((DH, D), lambda c: (0, 0))],
        out_specs=[pl.BlockSpec((PACK_TG, 128, 128), lambda c: (c, 0, 0)),
                   pl.BlockSpec((PACK_TG, 128, 128), lambda c: (c, 0, 0))],
        out_shape=[out_t, out_t],
    )(xT, yT, even_sel, odd_sel)
    m = nblk * PACK_TG * 128
    return jnp.reshape(outx, (m, 128)), jnp.reshape(outy, (m, 128))


def _sc_gather(user, item, language, category, utab4, itab4, ltabT, ctabT,
               item_languages, item_categories, wl, bl, wc, bc):
    f32 = jnp.float32
    i32 = jnp.int32
    mesh = plsc.VectorSubcoreMesh(core_axis_name="c", subcore_axis_name="s")

    @functools.partial(
        pl.kernel,
        out_type=[
            jax.ShapeDtypeStruct((D, B), f32),
            jax.ShapeDtypeStruct((D, B), f32),
            jax.ShapeDtypeStruct((B,), f32),
        ],
        mesh=mesh,
        compiler_params=pltpu.CompilerParams(use_tc_tiling_on_sc=True,
                                             needs_layout_passes=False),
        scratch_types=[
            pltpu.VMEM((BPW,), i32),
            pltpu.VMEM((BPW,), i32),
            pltpu.VMEM((BPW,), i32),
            pltpu.VMEM((BPW,), i32),
            pltpu.VMEM((BPW,), i32),
            pltpu.VMEM((BPW,), i32),
            pltpu.VMEM((BPW,), i32),
            pltpu.VMEM((BPW,), i32),
            pltpu.VMEM((BPW, 128), i32),
            pltpu.VMEM((D, BPW), f32),
            pltpu.VMEM((D, BPW), f32),
            pltpu.VMEM((DH, NL), f32),
            pltpu.VMEM((DH, NCAT), f32),
            pltpu.VMEM((DH,), f32),
            pltpu.VMEM((DH,), f32),
            pltpu.VMEM((L,), f32),
            pltpu.VMEM((L,), f32),
            pltpu.VMEM((BPW,), f32),
            pltpu.SemaphoreType.DMA,
            pltpu.SemaphoreType.DMA,
        ],
    )
    def sc_kernel(user_h, item_h, lang_h, cat_h, utab4_h, itab4_h, ltabT_h,
                  ctabT_h, ilang_h, icat_h, wl_h, bl_h, wc_h, bc_h,
                  uT_out, iT_out, gate_out,
                  uidx_v, iidx_v, lidx_v, cidx_v, ilidx_v, icidx_v,
                  uq_v, iq_v, x128_v, uT_v, iT_v, ltab_v, ctab_v,
                  wl_v, wc_v, bl_v, bc_v, gate_v, sem, sem2):
        wid = lax.axis_index("s") * NC + lax.axis_index("c")
        base = wid * BPW
        sl = pl.ds(base, BPW)
        pltpu.sync_copy(user_h.at[sl], uidx_v)
        pltpu.sync_copy(item_h.at[sl], iidx_v)
        pltpu.sync_copy(lang_h.at[sl], lidx_v)
        pltpu.sync_copy(cat_h.at[sl], cidx_v)
        m1 = pltpu.async_copy(ilang_h.at[iidx_v], ilidx_v, sem2)
        m2 = pltpu.async_copy(icat_h.at[iidx_v], icidx_v, sem2)
        pltpu.sync_copy(ltabT_h, ltab_v)
        pltpu.sync_copy(ctabT_h, ctab_v)
        pltpu.sync_copy(wl_h, wl_v)
        pltpu.sync_copy(wc_h, wc_v)
        pltpu.sync_copy(bl_h, bl_v)
        pltpu.sync_copy(bc_h, bc_v)

        def qbody(ci, _):
            s = pl.ds(ci * L, L)
            u = uidx_v[s]
            i = iidx_v[s]
            uq_v[s] = lax.shift_left(lax.shift_right_logical(u, 10), 7) \
                + (u & 127)
            iq_v[s] = lax.shift_left(lax.shift_right_logical(i, 10), 7) \
                + (i & 127)
            return ()

        lax.fori_loop(0, BPW // L, qbody, (), unroll=4)

        lane = lax.iota(i32, L)

        himask = jnp.int32(-65536)

        def extract(idx_ref, dst_ref):
            def ebody(ci, _):
                r0 = ci * L
                rows = r0 + lane
                basecol = (lax.shift_right_logical(idx_ref[pl.ds(r0, L)], 7)
                           & 7) * L
                for p in range(DH):
                    v = plsc.load_gather(x128_v, [rows, basecol + p])
                    dst_ref[2 * p, pl.ds(r0, L)] = \
                        plsc.bitcast(lax.shift_left(v, 16), f32)
                    dst_ref[2 * p + 1, pl.ds(r0, L)] = \
                        plsc.bitcast(v & himask, f32)
                return ()

            lax.fori_loop(0, BPW // L, ebody, (), unroll=2)

        pltpu.async_copy(utab4_h.at[uq_v], x128_v, sem).wait()
        extract(uidx_v, uT_v)
        pltpu.async_copy(itab4_h.at[iq_v], x128_v, sem).wait()
        extract(iidx_v, iT_v)

        m1.wait()
        m2.wait()

        wlvec = wl_v[...]
        wcvec = wc_v[...]
        blvec = bl_v[...]
        bcvec = bc_v[...]

        def chunk_body(ci, _):
            r0 = ci * L
            lidx = lidx_v[pl.ds(r0, L)]
            ilidx = ilidx_v[pl.ds(r0, L)]
            cidx = cidx_v[pl.ds(r0, L)]
            icidx = icidx_v[pl.ds(r0, L)]
            acc_l = jnp.zeros((L,), f32)
            acc_c = jnp.zeros((L,), f32)
            for d in range(DH):
                drow = jnp.full((L,), d, i32)
                lv = plsc.load_gather(ltab_v, [drow, lidx])
                ilv = plsc.load_gather(ltab_v, [drow, ilidx])
                acc_l = acc_l + jnp.abs(lv - ilv) * wlvec[d]
                cv = plsc.load_gather(ctab_v, [drow, cidx])
                icv = plsc.load_gather(ctab_v, [drow, icidx])
                acc_c = acc_c + jnp.abs(cv - icv) * wcvec[d]
            sig_l = 1.0 / (1.0 + jnp.exp(-(acc_l + blvec)))
            sig_c = 1.0 / (1.0 + jnp.exp(-(acc_c + bcvec)))
            gate_v[pl.ds(r0, L)] = sig_l * sig_c
            return ()

        lax.fori_loop(0, BPW // L, chunk_body, (), unroll=2)

        pltpu.sync_copy(uT_v, uT_out.at[:, sl])
        pltpu.sync_copy(iT_v, iT_out.at[:, sl])
        pltpu.sync_copy(gate_v, gate_out.at[sl])

    return sc_kernel(user, item, language, category, utab4, itab4, ltabT,
                     ctabT, item_languages, item_categories, wl, bl, wc, bc)


def _tc_dense(uT, iT, gate2d, W1uT, W1iT, b1c, W2T, b2c, w3c, b3):
    NB = 4096
    grid = (B // NB,)
    f32 = jnp.float32

    def body(uT_r, iT_r, gate_r, W1uT_r, W1iT_r, b1c_r, W2T_r, b2c_r,
             w3c_r, b3_r, out_r):
        h = jnp.dot(W1uT_r[...], uT_r[...], preferred_element_type=f32)
        h = h + jnp.dot(W1iT_r[...], iT_r[...], preferred_element_type=f32)
        h = jax.nn.relu(h + b1c_r[...])
        h = jax.nn.relu(jnp.dot(W2T_r[...], h, preferred_element_type=f32)
                        + b2c_r[...])
        base = jnp.sum(h * w3c_r[...], axis=0, keepdims=True) + b3_r[0, 0]
        out_r[...] = base * gate_r[...]

    colspec = lambda h: pl.BlockSpec((h, NB), lambda b: (0, b))
    full = lambda s: pl.BlockSpec(s, lambda b: (0,) * len(s))
    out = pl.pallas_call(
        body,
        grid=grid,
        in_specs=[
            colspec(D), colspec(D), colspec(1),
            full((128, D)), full((128, D)), full((128, 1)),
            full((64, 128)), full((64, 1)), full((64, 1)), full((1, 1)),
        ],
        out_specs=pl.BlockSpec((1, NB), lambda b: (0, b)),
        out_shape=jax.ShapeDtypeStruct((1, B), f32),
    )(uT, iT, gate2d, W1uT, W1iT, b1c, W2T, b2c, w3c, b3)
    return jnp.reshape(out, (B,))


def kernel(user, item, language, category, user_table, item_table,
           lang_table, cat_table, item_languages, item_categories,
           W_lang, b_lang, W_cat, b_cat, W1, b1, W2, b2, W3, b3):
    utab4, itab4 = _pack2(user_table.T, item_table.T)
    uT, iT, gate = _sc_gather(
        user, item, language, category, utab4, itab4,
        lang_table.T, cat_table.T, item_languages, item_categories,
        jnp.reshape(W_lang, (DH,)), jnp.broadcast_to(b_lang, (L,)),
        jnp.reshape(W_cat, (DH,)), jnp.broadcast_to(b_cat, (L,)))
    gate2d = jnp.reshape(gate, (1, B))
    W1uT = jnp.transpose(W1[:D])
    W1iT = jnp.transpose(W1[D:])
    b1c = jnp.reshape(b1, (128, 1))
    W2T = jnp.transpose(W2)
    b2c = jnp.reshape(b2, (64, 1))
    w3c = jnp.reshape(W3, (64, 1))
    b3c = jnp.reshape(b3, (1, 1))
    return _tc_dense(uT, iT, gate2d, W1uT, W1iT, b1c, W2T, b2c, w3c, b3c)
